# Initial kernel scaffold; baseline (speedup 1.0000x reference)
import jax
import jax.numpy as jnp
from jax.experimental import pallas as pl


def kernel(x, table, pos_encoding):
    raise NotImplementedError("write your pallas kernel here")


# speedup vs baseline: 1.3902x; 1.3902x over previous
import functools

import jax
import jax.numpy as jnp
from jax import lax
from jax.experimental import pallas as pl
from jax.experimental.pallas import tpu as pltpu
from jax.experimental.pallas import tpu_sc as plsc

_NC = 2
_NS = 16
_NW = _NC * _NS

_GROUP = 100
_CS = 4


def _make_sc_call(B, S, V, D):
    rows_per_w = (B // _NW) * S
    chunk_rows = _CS * S
    n_chunks = rows_per_w // chunk_rows
    groups_per_chunk = chunk_rows // _GROUP

    mesh = plsc.VectorSubcoreMesh(core_axis_name="c", subcore_axis_name="s")

    @functools.partial(
        pl.kernel,
        mesh=mesh,
        compiler_params=pltpu.CompilerParams(use_tc_tiling_on_sc=False),
        out_type=jax.ShapeDtypeStruct((B * S, D), jnp.float32),
        scratch_types=[
            pltpu.VMEM((groups_per_chunk, _GROUP), jnp.int32),
            pltpu.VMEM((chunk_rows, D), jnp.float32),
            pltpu.VMEM((S, D), jnp.float32),
            pltpu.SemaphoreType.DMA,
            pltpu.SemaphoreType.DMA,
        ],
    )
    def sc_call(idx_hbm, table_hbm, pos_hbm, out_hbm,
                idx_v, rows_v, pos_v, gsem, msem):
        wid = lax.axis_index("s") * _NC + lax.axis_index("c")
        row_base_w = wid * rows_per_w
        group_base_w = row_base_w // _GROUP

        pltpu.async_copy(pos_hbm, pos_v, msem).wait()

        def chunk_body(g, carry):
            row_base = pl.multiple_of(row_base_w + g * chunk_rows, chunk_rows)
            group_base = pl.multiple_of(
                group_base_w + g * groups_per_chunk, groups_per_chunk)

            pltpu.async_copy(
                idx_hbm.at[pl.ds(group_base, groups_per_chunk)], idx_v, msem
            ).wait()

            descs = []
            for j in range(groups_per_chunk):
                descs.append(pltpu.async_copy(
                    table_hbm.at[idx_v.at[j]],
                    rows_v.at[pl.ds(j * _GROUP, _GROUP)],
                    gsem,
                ))
            for dsc in descs:
                dsc.wait()

            def add_body(s, carry2):
                for half in range(D // 16):
                    p = pos_v[s, pl.ds(half * 16, 16)]
                    for q in range(_CS):
                        r = q * S + s
                        rows_v[r, pl.ds(half * 16, 16)] = (
                            rows_v[r, pl.ds(half * 16, 16)] + p)
                return carry2
            lax.fori_loop(0, S, add_body, 0)

            pltpu.async_copy(
                rows_v, out_hbm.at[pl.ds(row_base, chunk_rows)], msem
            ).wait()
            return carry

        lax.fori_loop(0, n_chunks, chunk_body, 0)

    return sc_call


def kernel(x, table, pos_encoding):
    B, S = x.shape
    V, D = table.shape
    idx2d = x.reshape(-1).reshape((B * S) // _GROUP, _GROUP)
    pos2d = pos_encoding[0, :S, :]
    out_flat = _make_sc_call(B, S, V, D)(idx2d, table, pos2d)
    return out_flat.reshape(B, S, D)
